# baseline (speedup 1.0000x reference)
import jax
import jax.numpy as jnp
from jax import lax
from jax.experimental import pallas as pl

NUM_TERMS = 1000000
D = 32
KREL = 32
NRELS = 40
B = 16384

CHUNKS = (8192, 8192)
assert sum(CHUNKS) == B


def _make_tc_body(cb):
    def _tc_body(lg_ref, rg_ref, rels_ref, w2_ref, g_ref, rv_ref, b_ref,
                 out_ref):
        lb = lg_ref[...]
        rb = rg_ref[...]
        t = jnp.dot(lb, w2_ref[...].astype(jnp.bfloat16),
                    preferred_element_type=jnp.float32
                    ).astype(jnp.bfloat16)
        rrep = jnp.concatenate([rb] * KREL, axis=1)
        p = t * rrep
        s = jnp.dot(p, g_ref[...].astype(jnp.bfloat16),
                    preferred_element_type=jnp.float32)
        ridx = rels_ref[...]
        onehot = (lax.broadcasted_iota(jnp.int32, (cb, NRELS), 1) == ridx
                  ).astype(jnp.float32)
        biascol = jnp.dot(rv_ref[...], b_ref[...],
                          preferred_element_type=jnp.float32)
        energy = (jnp.sum(s * onehot, axis=1, keepdims=True)
                  + jnp.dot(onehot, biascol,
                            preferred_element_type=jnp.float32))
        out_ref[...] = energy
    return _tc_body


def _tc_score(cb, rows, rels2d, w2, g, rel_vecs, b2):
    return pl.pallas_call(
        _make_tc_body(cb),
        grid=(1,),
        in_specs=[
            pl.BlockSpec((cb, D), lambda i: (0, 0)),
            pl.BlockSpec((cb, D), lambda i: (1, 0)),
            pl.BlockSpec((cb, 1), lambda i: (0, 0)),
            pl.BlockSpec((D, KREL * D), lambda i: (0, 0)),
            pl.BlockSpec((KREL * D, NRELS), lambda i: (0, 0)),
            pl.BlockSpec((NRELS, KREL), lambda i: (0, 0)),
            pl.BlockSpec((KREL, 1), lambda i: (0, 0)),
        ],
        out_specs=pl.BlockSpec((cb, 1), lambda i: (0, 0)),
        out_shape=jax.ShapeDtypeStruct((cb, 1), jnp.float32),
    )(rows, rows, rels2d, w2, g, rel_vecs, b2)


def kernel(term_vecs, rel_vecs, assoc_W, assoc_b, rels, terms_L, terms_R):
    w2 = assoc_W.transpose(1, 0, 2).reshape(D, KREL * D)
    g = jnp.repeat(rel_vecs.T, D, axis=0)
    b2 = assoc_b.reshape(KREL, 1)
    rels2d = rels.astype(jnp.int32).reshape(B, 1)

    outs = []
    base = 0
    prev_rows = None
    for cb in CHUNKS:
        sl = slice(base, base + cb)
        base += cb
        idx_c = jnp.concatenate([terms_L[sl], terms_R[sl]])
        rows_c = term_vecs.at[idx_c].get(
            mode="promise_in_bounds").astype(jnp.bfloat16)
        outs.append(_tc_score(cb, rows_c, rels2d[sl], w2, g, rel_vecs, b2))
    return jnp.concatenate(outs, axis=0).reshape(B)


# speedup vs baseline: 6.9758x; 6.9758x over previous
import jax
import jax.numpy as jnp
from jax import lax
from jax.experimental import pallas as pl

NUM_TERMS = 1000000
D = 32
KREL = 32
NRELS = 40
B = 16384

CHUNKS = (8192, 8192)
assert sum(CHUNKS) == B


def _make_tc_body(cb):
    def _tc_body(lg_ref, rg_ref, rels_ref, w2_ref, g_ref, rv_ref, b_ref,
                 out_ref):
        lb = lg_ref[...].astype(jnp.bfloat16)
        rb = rg_ref[...].astype(jnp.bfloat16)
        t = jnp.dot(lb, w2_ref[...].astype(jnp.bfloat16),
                    preferred_element_type=jnp.float32
                    ).astype(jnp.bfloat16)
        rrep = jnp.concatenate([rb] * KREL, axis=1)
        p = t * rrep
        s = jnp.dot(p, g_ref[...].astype(jnp.bfloat16),
                    preferred_element_type=jnp.float32)
        ridx = rels_ref[...]
        onehot = (lax.broadcasted_iota(jnp.int32, (cb, NRELS), 1) == ridx
                  ).astype(jnp.float32)
        biascol = jnp.dot(rv_ref[...], b_ref[...],
                          preferred_element_type=jnp.float32)
        energy = (jnp.sum(s * onehot, axis=1, keepdims=True)
                  + jnp.dot(onehot, biascol,
                            preferred_element_type=jnp.float32))
        out_ref[...] = energy
    return _tc_body


def _tc_score(cb, rows, rels2d, w2, g, rel_vecs, b2):
    return pl.pallas_call(
        _make_tc_body(cb),
        grid=(1,),
        in_specs=[
            pl.BlockSpec((cb, D), lambda i: (0, 0)),
            pl.BlockSpec((cb, D), lambda i: (1, 0)),
            pl.BlockSpec((cb, 1), lambda i: (0, 0)),
            pl.BlockSpec((D, KREL * D), lambda i: (0, 0)),
            pl.BlockSpec((KREL * D, NRELS), lambda i: (0, 0)),
            pl.BlockSpec((NRELS, KREL), lambda i: (0, 0)),
            pl.BlockSpec((KREL, 1), lambda i: (0, 0)),
        ],
        out_specs=pl.BlockSpec((cb, 1), lambda i: (0, 0)),
        out_shape=jax.ShapeDtypeStruct((cb, 1), jnp.float32),
    )(rows, rows, rels2d, w2, g, rel_vecs, b2)


def kernel(term_vecs, rel_vecs, assoc_W, assoc_b, rels, terms_L, terms_R):
    w2 = assoc_W.transpose(1, 0, 2).reshape(D, KREL * D)
    g = jnp.repeat(rel_vecs.T, D, axis=0)
    b2 = assoc_b.reshape(KREL, 1)
    rels2d = rels.astype(jnp.int32).reshape(B, 1)

    outs = []
    base = 0
    prev_rows = None
    for cb in CHUNKS:
        sl = slice(base, base + cb)
        base += cb
        idx_c = jnp.concatenate([terms_L[sl], terms_R[sl]])
        rows_c = term_vecs.at[idx_c].get(mode="promise_in_bounds")
        outs.append(_tc_score(cb, rows_c, rels2d[sl], w2, g, rel_vecs, b2))
    return jnp.concatenate(outs, axis=0).reshape(B)
